# baseline (speedup 1.0000x reference)
import functools

import jax
import jax.numpy as jnp
from jax import lax
from jax.experimental import pallas as pl
from jax.experimental.pallas import tpu as pltpu
from jax.experimental.pallas import tpu_sc as plsc

B, S, D = 4, 8192, 1024
NC, NS = 2, 16
NW = NC * NS
TOT = B * S * D
EPW = TOT // NW
CE = 8192
NCHUNK = EPW // CE
NBUF = 4
VEC = 16


def kernel(inputs, pos_table):
    x1 = inputs.reshape(TOT)
    t1 = pos_table.reshape(S * D)
    mesh = plsc.VectorSubcoreMesh(core_axis_name="c", subcore_axis_name="s")

    @functools.partial(
        pl.kernel,
        mesh=mesh,
        out_type=jax.ShapeDtypeStruct((TOT,), jnp.float32),
        scratch_types=(
            [pltpu.VMEM((CE,), jnp.float32) for _ in range(NBUF)]
            + [pltpu.VMEM((CE,), jnp.float32) for _ in range(NBUF)]
            + [pltpu.SemaphoreType.DMA((NBUF,)), pltpu.SemaphoreType.DMA((NBUF,))]
    ),
    )
    def sc_add(x_hbm, t_hbm, o_hbm, *scratch):
        xb = scratch[:NBUF]
        ob = scratch[NBUF:2 * NBUF]
        in_sem, out_sem = scratch[2 * NBUF], scratch[2 * NBUF + 1]

        wid = lax.axis_index("s") * NC + lax.axis_index("c")
        base = wid * EPW
        tbase = base % (S * D)

        def issue_in(c, b):
            pltpu.async_copy(x_hbm.at[pl.ds(base + c * CE, CE)], xb[b], in_sem.at[b])
            pltpu.async_copy(t_hbm.at[pl.ds(tbase + c * CE, CE)], ob[b], in_sem.at[b])

        def wait_in(c, b):
            pltpu.make_async_copy(x_hbm.at[pl.ds(base + c * CE, CE)], xb[b], in_sem.at[b]).wait()
            pltpu.make_async_copy(t_hbm.at[pl.ds(tbase + c * CE, CE)], ob[b], in_sem.at[b]).wait()

        def issue_out(c, b):
            pltpu.async_copy(ob[b], o_hbm.at[pl.ds(base + c * CE, CE)], out_sem.at[b])

        def wait_out(c, b):
            pltpu.make_async_copy(ob[b], o_hbm.at[pl.ds(base + c * CE, CE)], out_sem.at[b]).wait()

        def compute(b):
            def kbody(k, carry):
                off = k * VEC
                plsc.addupdate(ob[b].at[pl.ds(off, VEC)], xb[b][pl.ds(off, VEC)])
                return carry
            lax.fori_loop(0, CE // VEC, kbody, 0)

        def step(c, b):
            wait_in(c, b)
            compute(b)
            issue_out(c, b)

        for b in range(NBUF):
            issue_in(b, b)

        for b in range(NBUF):
            c = b
            step(c, b)
            if c >= 1:
                wait_out(c - 1, (b - 1) % NBUF)
                issue_in(c + 3, (b - 1) % NBUF)

        def gbody(g, carry):
            for b in range(NBUF):
                c = g * NBUF + b
                step(c, b)
                wait_out(c - 1, (b - 1) % NBUF)
                issue_in(c + 3, (b - 1) % NBUF)
            return carry

        lax.fori_loop(1, NCHUNK // NBUF - 1, gbody, 0)

        g = NCHUNK // NBUF - 1
        for b in range(NBUF):
            c = g * NBUF + b
            step(c, b)
            if b == 0:
                wait_out(c - 1, (b - 1) % NBUF)
                issue_in(c + 3, (b - 1) % NBUF)

        for b in range(NBUF):
            c = (NCHUNK - NBUF) + b
            wait_out(c, b)

    return sc_add(x1, t1).reshape(B, S, D)


# speedup vs baseline: 1.2972x; 1.2972x over previous
import functools

import jax
import jax.numpy as jnp
from jax import lax
from jax.experimental import pallas as pl
from jax.experimental.pallas import tpu as pltpu
from jax.experimental.pallas import tpu_sc as plsc

B, S, D = 4, 8192, 1024
NC, NS = 2, 16
NW = NC * NS
TOT = B * S * D
EPW = TOT // NW
CE = 8192
NCHUNK = EPW // CE
NBUF = 4
VEC = 16


def kernel(inputs, pos_table):
    x1 = inputs.reshape(TOT)
    t1 = pos_table.reshape(S * D)
    mesh = plsc.VectorSubcoreMesh(core_axis_name="c", subcore_axis_name="s")

    @functools.partial(
        pl.kernel,
        mesh=mesh,
        out_type=jax.ShapeDtypeStruct((TOT,), jnp.float32),
        scratch_types=(
            [pltpu.VMEM((CE,), jnp.float32) for _ in range(NBUF)]
            + [pltpu.VMEM((CE,), jnp.float32) for _ in range(NBUF)]
            + [pltpu.SemaphoreType.DMA((NBUF,)), pltpu.SemaphoreType.DMA((NBUF,))]
    ),
    )
    def sc_add(x_hbm, t_hbm, o_hbm, *scratch):
        xb = scratch[:NBUF]
        ob = scratch[NBUF:2 * NBUF]
        in_sem, out_sem = scratch[2 * NBUF], scratch[2 * NBUF + 1]

        wid = lax.axis_index("s") * NC + lax.axis_index("c")
        base = wid * EPW
        tbase = base % (S * D)

        def issue_in(c, b):
            pltpu.async_copy(x_hbm.at[pl.ds(base + c * CE, CE)], xb[b], in_sem.at[b])
            pltpu.async_copy(t_hbm.at[pl.ds(tbase + c * CE, CE)], ob[b], in_sem.at[b])

        def wait_in(c, b):
            pltpu.make_async_copy(x_hbm.at[pl.ds(base + c * CE, CE)], xb[b], in_sem.at[b]).wait()
            pltpu.make_async_copy(t_hbm.at[pl.ds(tbase + c * CE, CE)], ob[b], in_sem.at[b]).wait()

        def issue_out(c, b):
            pltpu.async_copy(ob[b], o_hbm.at[pl.ds(base + c * CE, CE)], out_sem.at[b])

        def wait_out(c, b):
            pltpu.make_async_copy(ob[b], o_hbm.at[pl.ds(base + c * CE, CE)], out_sem.at[b]).wait()

        def compute(b):
            UNROLL = 16

            def kbody(k, carry):
                for u in range(UNROLL):
                    off = (k * UNROLL + u) * VEC
                    plsc.addupdate(ob[b].at[pl.ds(off, VEC)], xb[b][pl.ds(off, VEC)])
                return carry
            lax.fori_loop(0, CE // (VEC * UNROLL), kbody, 0)

        def step(c, b):
            wait_in(c, b)
            compute(b)
            issue_out(c, b)

        for b in range(NBUF):
            issue_in(b, b)

        for b in range(NBUF):
            c = b
            step(c, b)
            if c >= 1:
                wait_out(c - 1, (b - 1) % NBUF)
                issue_in(c + 3, (b - 1) % NBUF)

        def gbody(g, carry):
            for b in range(NBUF):
                c = g * NBUF + b
                step(c, b)
                wait_out(c - 1, (b - 1) % NBUF)
                issue_in(c + 3, (b - 1) % NBUF)
            return carry

        lax.fori_loop(1, NCHUNK // NBUF - 1, gbody, 0)

        g = NCHUNK // NBUF - 1
        for b in range(NBUF):
            c = g * NBUF + b
            step(c, b)
            if b == 0:
                wait_out(c - 1, (b - 1) % NBUF)
                issue_in(c + 3, (b - 1) % NBUF)

        for b in range(NBUF):
            c = (NCHUNK - NBUF) + b
            wait_out(c, b)

    return sc_add(x1, t1).reshape(B, S, D)


# speedup vs baseline: 2.1648x; 1.6688x over previous
import functools

import jax
import jax.numpy as jnp
from jax import lax
from jax.experimental import pallas as pl
from jax.experimental.pallas import tpu as pltpu
from jax.experimental.pallas import tpu_sc as plsc

B, S, D = 4, 8192, 1024
NC, NS = 2, 16
NW = NC * NS
RPW = (B * S) // NW
WPB = S // RPW
CR = 16
NCHUNK = RPW // CR
NBUF = 3
VEC = 16
UNROLL = 16


def kernel(inputs, pos_table):
    mesh = plsc.VectorSubcoreMesh(core_axis_name="c", subcore_axis_name="s")

    @functools.partial(
        pl.kernel,
        mesh=mesh,
        out_type=jax.ShapeDtypeStruct((B, S, D), jnp.float32),
        scratch_types=(
            [pltpu.VMEM((CR, D), jnp.float32) for _ in range(2 * NBUF)]
            + [pltpu.SemaphoreType.DMA((NBUF,)), pltpu.SemaphoreType.DMA((NBUF,))]
        ),
    )
    def sc_add(x_hbm, t_hbm, o_hbm, *scratch):
        xb = scratch[:NBUF]
        ob = scratch[NBUF:2 * NBUF]
        in_sem, out_sem = scratch[2 * NBUF], scratch[2 * NBUF + 1]

        wid = lax.axis_index("s") * NC + lax.axis_index("c")
        bat = wid // WPB
        row0 = (wid % WPB) * RPW

        def issue_in(c, b):
            r = row0 + c * CR
            pltpu.async_copy(x_hbm.at[bat, pl.ds(r, CR)], xb[b], in_sem.at[b])
            pltpu.async_copy(t_hbm.at[pl.ds(r, CR)], ob[b], in_sem.at[b])

        def wait_in(c, b):
            r = row0 + c * CR
            pltpu.make_async_copy(x_hbm.at[bat, pl.ds(r, CR)], xb[b], in_sem.at[b]).wait()
            pltpu.make_async_copy(t_hbm.at[pl.ds(r, CR)], ob[b], in_sem.at[b]).wait()

        def issue_out(c, b):
            r = row0 + c * CR
            pltpu.async_copy(ob[b], o_hbm.at[bat, pl.ds(r, CR)], out_sem.at[b])

        def wait_out(c, b):
            r = row0 + c * CR
            pltpu.make_async_copy(ob[b], o_hbm.at[bat, pl.ds(r, CR)], out_sem.at[b]).wait()

        def compute(b):
            def kbody(k, carry):
                for u in range(UNROLL):
                    idx = k * UNROLL + u
                    r = idx // (D // VEC)
                    col = (idx % (D // VEC)) * VEC
                    plsc.addupdate(ob[b].at[r, pl.ds(col, VEC)],
                                   xb[b][r, pl.ds(col, VEC)])
                return carry
            lax.fori_loop(0, (CR * D) // (VEC * UNROLL), kbody, 0)

        def step(c, b):
            wait_in(c, b)
            compute(b)
            issue_out(c, b)

        for b in range(NBUF):
            issue_in(b, b)

        for c in range(NBUF):
            step(c, c)
            if c >= 1:
                wait_out(c - 1, (c - 1) % NBUF)
                issue_in(c + 2, (c + 2) % NBUF)

        def gbody(g, carry):
            for b in range(NBUF):
                c = g * NBUF + b
                step(c, b)
                wait_out(c - 1, (b + 2) % NBUF)
                @pl.when(c + 2 < NCHUNK)
                def _():
                    issue_in(c + 2, (b + 2) % NBUF)
            return carry

        lax.fori_loop(1, (NCHUNK - 1) // NBUF, gbody, 0)

        c = NCHUNK - 1
        step(c, c % NBUF)
        wait_out(c - 1, (c - 1) % NBUF)

        wait_out(c, c % NBUF)

    return sc_add(inputs, pos_table)
